# baseline (speedup 1.0000x reference)
import functools

import jax
import jax.numpy as jnp
from jax import lax
from jax.experimental import pallas as pl
from jax.experimental.pallas import tpu as pltpu
from jax.experimental.pallas import tpu_sc as plsc

VOCAB = 1000000
EMBED = 64
BATCH = 4096
SEQ = 200
LANES = 128

_info = plsc.get_sparse_core_info()
NC = _info.num_cores
NS = _info.num_subcores
NW = NC * NS

FULL_BLOCKS = VOCAB // LANES
TAIL = VOCAB - FULL_BLOCKS * LANES
STEPS_A = (FULL_BLOCKS + NW - 1) // NW

BBLK = BATCH // NW
assert BBLK == 128


def _iota16():
    return lax.iota(jnp.int32, 16)


def _mesh():
    return plsc.VectorSubcoreMesh(core_axis_name="c", subcore_axis_name="s")


def _transpose_table(table_t, tail_p):
    @functools.partial(
        pl.kernel,
        mesh=_mesh(),
        out_type=jax.ShapeDtypeStruct((VOCAB, LANES), jnp.float32),
        scratch_types=[
            *[pltpu.VMEM((EMBED, LANES), jnp.float32) for _ in range(2)],
            *[pltpu.VMEM((LANES, LANES), jnp.float32) for _ in range(2)],
            *[pltpu.SemaphoreType.DMA for _ in range(4)],
        ],
        compiler_params=pltpu.CompilerParams(needs_layout_passes=False),
    )
    def k(t_hbm, tail_hbm, tpad_hbm, in0, in1, out0, out1, is0, is1,
          os0, os1):
        ins, outs = (in0, in1), (out0, out1)
        isems, osems = (is0, is1), (os0, os1)
        wid = lax.axis_index("s") * NC + lax.axis_index("c")

        def v0_of(i):
            b = wid + NW * i
            return jnp.where(b < FULL_BLOCKS, b * LANES,
                             (FULL_BLOCKS - 1) * LANES)

        def start_read(i, b):
            pltpu.async_copy(t_hbm.at[:, pl.ds(v0_of(i), LANES)], ins[b],
                             isems[b])

        def wait_read(b):
            pltpu.make_async_copy(t_hbm.at[:, pl.ds(0, LANES)], ins[b],
                                  isems[b]).wait()

        def transpose(src, dst):
            rows = [_iota16() + 16 * j for j in range(EMBED // 16)]
            for v in range(LANES):
                col = jnp.full((16,), v, jnp.int32)
                for j in range(EMBED // 16):
                    dst[v, pl.ds(16 * j, 16)] = plsc.load_gather(
                        src, [rows[j], col])

        def start_write(i, b):
            pltpu.async_copy(outs[b], tpad_hbm.at[pl.ds(v0_of(i), LANES), :],
                             osems[b])

        def wait_write(b):
            pltpu.make_async_copy(
                outs[b], tpad_hbm.at[pl.ds(0, LANES), :], osems[b]).wait()

        start_read(0, 0)
        start_read(1, 1)

        def body(g, carry):
            for b in range(2):
                i = 2 * g + b
                wait_read(b)

                @pl.when(i >= 2)
                def _():
                    wait_write(b)

                transpose(ins[b], outs[b])
                start_write(i, b)

                @pl.when(i + 2 < STEPS_A)
                def _():
                    start_read(i + 2, b)

            return carry

        lax.fori_loop(0, STEPS_A // 2, body, 0)

        i = STEPS_A - 1
        b = i % 2
        wait_read(b)
        wait_write(b)
        transpose(ins[b], outs[b])
        start_write(i, b)
        wait_write(1 - b)
        wait_write(b)

        @pl.when(wid == 0)
        def _():
            tbuf = out0.at[pl.ds(0, TAIL), :]
            pltpu.async_copy(tail_hbm, tbuf, os0)
            pltpu.make_async_copy(tail_hbm, tbuf, os0).wait()
            pltpu.async_copy(
                tbuf, tpad_hbm.at[pl.ds(FULL_BLOCKS * LANES, TAIL), :], os0)
            pltpu.make_async_copy(
                tbuf, tpad_hbm.at[pl.ds(0, TAIL), :], os0).wait()

    return k(table_t, tail_p)


def _gather(tpad, idx_t):
    @functools.partial(
        pl.kernel,
        mesh=_mesh(),
        out_type=jax.ShapeDtypeStruct((SEQ, EMBED, BATCH), jnp.float32),
        scratch_types=[
            pltpu.VMEM((SEQ, BBLK), jnp.int32),
            *[pltpu.VMEM((BBLK, LANES), jnp.float32) for _ in range(3)],
            *[pltpu.VMEM((EMBED, BBLK), jnp.float32) for _ in range(3)],
            pltpu.SemaphoreType.DMA,
            *[pltpu.SemaphoreType.DMA for _ in range(3)],
            *[pltpu.SemaphoreType.DMA for _ in range(3)],
        ],
        compiler_params=pltpu.CompilerParams(needs_layout_passes=False),
    )
    def k(tpad_hbm, idx_hbm, out_hbm, idx_v, g0, g1, g2, o0, o1, o2,
          xsem, gs0, gs1, gs2, os0, os1, os2):
        gbufs, gsems = (g0, g1, g2), (gs0, gs1, gs2)
        obufs, osems = (o0, o1, o2), (os0, os1, os2)
        wid = lax.axis_index("s") * NC + lax.axis_index("c")
        b0 = wid * BBLK

        pltpu.async_copy(idx_hbm.at[:, pl.ds(b0, BBLK)], idx_v, xsem)
        pltpu.make_async_copy(
            idx_hbm.at[:, pl.ds(0, BBLK)], idx_v, xsem).wait()

        def start_gather(s, g):
            pltpu.async_copy(tpad_hbm.at[idx_v.at[s]], gbufs[g], gsems[g])

        def wait_gather(s, g):
            pltpu.make_async_copy(
                tpad_hbm.at[idx_v.at[s]], gbufs[g], gsems[g]).wait()

        def transpose(src, dst):
            rows = [_iota16() + 16 * j for j in range(BBLK // 16)]
            for e in range(EMBED):
                col = jnp.full((16,), e, jnp.int32)
                for j in range(BBLK // 16):
                    dst[e, pl.ds(16 * j, 16)] = plsc.load_gather(
                        src, [rows[j], col])

        def start_out(s, o):
            pltpu.async_copy(obufs[o], out_hbm.at[s, :, pl.ds(b0, BBLK)],
                             osems[o])

        def wait_out(o):
            pltpu.make_async_copy(
                obufs[o], out_hbm.at[0, :, pl.ds(b0, BBLK)], osems[o]).wait()

        for s in range(3):
            start_gather(s, s)

        def body(g3, carry):
            for k3 in range(3):
                s = 3 * g3 + k3
                wait_gather(s, k3)

                @pl.when(s >= 3)
                def _():
                    wait_out(k3)

                transpose(gbufs[k3], obufs[k3])
                start_out(s, k3)

                @pl.when(s + 3 < SEQ)
                def _():
                    start_gather(s + 3, k3)

            return carry

        lax.fori_loop(0, SEQ // 3, body, 0)

        for s in (SEQ - 2, SEQ - 1):
            k3 = s % 3
            wait_gather(s, k3)
            wait_out(k3)
            transpose(gbufs[k3], obufs[k3])
            start_out(s, k3)
        for k3 in range(3):
            wait_out(k3)

    return k(tpad, idx_t)


def kernel(input, table):
    tail_p = jnp.pad(table[FULL_BLOCKS * LANES:], ((0, 0), (0, LANES - EMBED)))
    tpad = _transpose_table(table.T, tail_p)
    out_t = _gather(tpad, input.T)
    return jnp.transpose(out_t, (2, 0, 1))


# speedup vs baseline: 1.6066x; 1.6066x over previous
import functools

import jax
import jax.numpy as jnp
from jax import lax
from jax.experimental import pallas as pl
from jax.experimental.pallas import tpu as pltpu
from jax.experimental.pallas import tpu_sc as plsc

VOCAB = 1000000
EMBED = 64
BATCH = 4096
SEQ = 200
LANES = 128

_info = plsc.get_sparse_core_info()
NC = _info.num_cores
NS = _info.num_subcores
NW = NC * NS

FULL_BLOCKS = VOCAB // LANES
TAIL = VOCAB - FULL_BLOCKS * LANES
STEPS_A = (FULL_BLOCKS + NW - 1) // NW

BBLK = BATCH // NW
assert BBLK == 128


def _iota16():
    return lax.iota(jnp.int32, 16)


def _mesh():
    return plsc.VectorSubcoreMesh(core_axis_name="c", subcore_axis_name="s")


def _transpose_table(table_t, tail_p):
    @functools.partial(
        pl.kernel,
        mesh=_mesh(),
        out_type=jax.ShapeDtypeStruct((VOCAB, LANES), jnp.float32),
        scratch_types=[
            *[pltpu.VMEM((EMBED, LANES), jnp.float32) for _ in range(2)],
            *[pltpu.VMEM((LANES, LANES), jnp.float32) for _ in range(2)],
            *[pltpu.SemaphoreType.DMA for _ in range(4)],
        ],
        compiler_params=pltpu.CompilerParams(needs_layout_passes=False),
    )
    def k(t_hbm, tail_hbm, tpad_hbm, in0, in1, out0, out1, is0, is1,
          os0, os1):
        ins, outs = (in0, in1), (out0, out1)
        isems, osems = (is0, is1), (os0, os1)
        wid = lax.axis_index("s") * NC + lax.axis_index("c")

        def v0_of(i):
            b = wid + NW * i
            return jnp.where(b < FULL_BLOCKS, b * LANES,
                             (FULL_BLOCKS - 1) * LANES)

        def start_read(i, b):
            pltpu.async_copy(t_hbm.at[:, pl.ds(v0_of(i), LANES)], ins[b],
                             isems[b])

        def wait_read(b):
            pltpu.make_async_copy(t_hbm.at[:, pl.ds(0, LANES)], ins[b],
                                  isems[b]).wait()

        rows_a = [_iota16() + 16 * j for j in range(EMBED // 16)]

        def transpose(src, dst):
            @plsc.parallel_loop(0, LANES, unroll=4)
            def _(v):
                col = jnp.full((16,), v, jnp.int32)
                vs = [plsc.load_gather(src, [rows_a[j], col])
                      for j in range(EMBED // 16)]
                for j in range(EMBED // 16):
                    plsc.store_scatter(dst, [col, rows_a[j]], vs[j])

        def start_write(i, b):
            pltpu.async_copy(outs[b], tpad_hbm.at[pl.ds(v0_of(i), LANES), :],
                             osems[b])

        def wait_write(b):
            pltpu.make_async_copy(
                outs[b], tpad_hbm.at[pl.ds(0, LANES), :], osems[b]).wait()

        start_read(0, 0)
        start_read(1, 1)

        def body(g, carry):
            for b in range(2):
                i = 2 * g + b
                wait_read(b)

                @pl.when(i >= 2)
                def _():
                    wait_write(b)

                transpose(ins[b], outs[b])
                start_write(i, b)

                @pl.when(i + 2 < STEPS_A)
                def _():
                    start_read(i + 2, b)

            return carry

        lax.fori_loop(0, STEPS_A // 2, body, 0)

        i = STEPS_A - 1
        b = i % 2
        wait_read(b)
        wait_write(b)
        transpose(ins[b], outs[b])
        start_write(i, b)
        wait_write(1 - b)
        wait_write(b)

        @pl.when(wid == 0)
        def _():
            tbuf = out0.at[pl.ds(0, TAIL), :]
            pltpu.async_copy(tail_hbm, tbuf, os0)
            pltpu.make_async_copy(tail_hbm, tbuf, os0).wait()
            pltpu.async_copy(
                tbuf, tpad_hbm.at[pl.ds(FULL_BLOCKS * LANES, TAIL), :], os0)
            pltpu.make_async_copy(
                tbuf, tpad_hbm.at[pl.ds(0, TAIL), :], os0).wait()

    return k(table_t, tail_p)


def _gather(tpad, idx_t):
    @functools.partial(
        pl.kernel,
        mesh=_mesh(),
        out_type=jax.ShapeDtypeStruct((SEQ, EMBED, BATCH), jnp.float32),
        scratch_types=[
            pltpu.VMEM((SEQ, BBLK), jnp.int32),
            *[pltpu.VMEM((BBLK, LANES), jnp.float32) for _ in range(3)],
            *[pltpu.VMEM((EMBED, BBLK), jnp.float32) for _ in range(3)],
            pltpu.SemaphoreType.DMA,
            *[pltpu.SemaphoreType.DMA for _ in range(3)],
            *[pltpu.SemaphoreType.DMA for _ in range(3)],
        ],
        compiler_params=pltpu.CompilerParams(needs_layout_passes=False),
    )
    def k(tpad_hbm, idx_hbm, out_hbm, idx_v, g0, g1, g2, o0, o1, o2,
          xsem, gs0, gs1, gs2, os0, os1, os2):
        gbufs, gsems = (g0, g1, g2), (gs0, gs1, gs2)
        obufs, osems = (o0, o1, o2), (os0, os1, os2)
        wid = lax.axis_index("s") * NC + lax.axis_index("c")
        b0 = wid * BBLK

        pltpu.async_copy(idx_hbm.at[:, pl.ds(b0, BBLK)], idx_v, xsem)
        pltpu.make_async_copy(
            idx_hbm.at[:, pl.ds(0, BBLK)], idx_v, xsem).wait()

        def start_gather(s, g):
            pltpu.async_copy(tpad_hbm.at[idx_v.at[s]], gbufs[g], gsems[g])

        def wait_gather(s, g):
            pltpu.make_async_copy(
                tpad_hbm.at[idx_v.at[s]], gbufs[g], gsems[g]).wait()

        rows_b = [_iota16() + 16 * j for j in range(BBLK // 16)]

        def transpose(src, dst):
            @plsc.parallel_loop(0, EMBED, unroll=4)
            def _(e):
                col = jnp.full((16,), e, jnp.int32)
                vs = [plsc.load_gather(src, [rows_b[j], col])
                      for j in range(BBLK // 16)]
                for j in range(BBLK // 16):
                    plsc.store_scatter(dst, [col, rows_b[j]], vs[j])

        def start_out(s, o):
            pltpu.async_copy(obufs[o], out_hbm.at[s, :, pl.ds(b0, BBLK)],
                             osems[o])

        def wait_out(o):
            pltpu.make_async_copy(
                obufs[o], out_hbm.at[0, :, pl.ds(b0, BBLK)], osems[o]).wait()

        for s in range(3):
            start_gather(s, s)

        def body(g3, carry):
            for k3 in range(3):
                s = 3 * g3 + k3
                wait_gather(s, k3)

                @pl.when(s >= 3)
                def _():
                    wait_out(k3)

                transpose(gbufs[k3], obufs[k3])
                start_out(s, k3)

                @pl.when(s + 3 < SEQ)
                def _():
                    start_gather(s + 3, k3)

            return carry

        lax.fori_loop(0, SEQ // 3, body, 0)

        for s in (SEQ - 2, SEQ - 1):
            k3 = s % 3
            wait_gather(s, k3)
            wait_out(k3)
            transpose(gbufs[k3], obufs[k3])
            start_out(s, k3)
        for k3 in range(3):
            wait_out(k3)

    return k(tpad, idx_t)


def kernel(input, table):
    tail_p = jnp.pad(table[FULL_BLOCKS * LANES:], ((0, 0), (0, LANES - EMBED)))
    tpad = _transpose_table(table.T, tail_p)
    out_t = _gather(tpad, input.T)
    return jnp.transpose(out_t, (2, 0, 1))


# speedup vs baseline: 4.4679x; 2.7809x over previous
import functools

import numpy as np

import jax
import jax.numpy as jnp
from jax import lax
from jax.experimental import pallas as pl
from jax.experimental.pallas import tpu as pltpu
from jax.experimental.pallas import tpu_sc as plsc

VOCAB = 1000000
EMBED = 64
BATCH = 4096
SEQ = 200
LANES = 128

_info = plsc.get_sparse_core_info()
NC = _info.num_cores
NS = _info.num_subcores
NW = NC * NS

FULL_BLOCKS = VOCAB // LANES
TAIL = VOCAB - FULL_BLOCKS * LANES
STEPS_A = (FULL_BLOCKS + NW - 1) // NW

BBLK = BATCH // NW
assert BBLK == 128


def _iota16():
    return lax.iota(jnp.int32, 16)


def _mesh():
    return plsc.VectorSubcoreMesh(core_axis_name="c", subcore_axis_name="s")


def _transpose_table(table_t, tail_p):
    @functools.partial(
        pl.kernel,
        mesh=_mesh(),
        out_type=jax.ShapeDtypeStruct((VOCAB, LANES), jnp.float32),
        scratch_types=[
            *[pltpu.VMEM((EMBED, LANES), jnp.float32) for _ in range(2)],
            *[pltpu.VMEM((LANES, LANES), jnp.float32) for _ in range(2)],
            *[pltpu.SemaphoreType.DMA for _ in range(4)],
        ],
        compiler_params=pltpu.CompilerParams(needs_layout_passes=False),
    )
    def k(t_hbm, tail_hbm, tpad_hbm, in0, in1, out0, out1, is0, is1,
          os0, os1):
        ins, outs = (in0, in1), (out0, out1)
        isems, osems = (is0, is1), (os0, os1)
        wid = lax.axis_index("s") * NC + lax.axis_index("c")

        def v0_of(i):
            b = wid + NW * i
            return jnp.where(b < FULL_BLOCKS, b * LANES,
                             (FULL_BLOCKS - 1) * LANES)

        def start_read(i, b):
            pltpu.async_copy(t_hbm.at[:, pl.ds(v0_of(i), LANES)], ins[b],
                             isems[b])

        def wait_read(b):
            pltpu.make_async_copy(t_hbm.at[:, pl.ds(0, LANES)], ins[b],
                                  isems[b]).wait()

        rows_a = [_iota16() + 16 * j for j in range(EMBED // 16)]
        i16 = _iota16()

        def transpose(src, dst):
            for je in range(EMBED // 16):
                rowv = rows_a[je]

                @plsc.parallel_loop(0, LANES, step=16, unroll=2)
                def _(v0):
                    for g in range(16):
                        colv = jnp.where(i16 < 16 - g, i16 + g,
                                         i16 + (g - 16)) + v0
                        val = plsc.load_gather(src, [rowv, colv])
                        plsc.store_scatter(dst, [colv, rowv], val)

        def start_write(i, b):
            pltpu.async_copy(outs[b], tpad_hbm.at[pl.ds(v0_of(i), LANES), :],
                             osems[b])

        def wait_write(b):
            pltpu.make_async_copy(
                outs[b], tpad_hbm.at[pl.ds(0, LANES), :], osems[b]).wait()

        start_read(0, 0)
        start_read(1, 1)

        def body(g, carry):
            for b in range(2):
                i = 2 * g + b
                wait_read(b)

                @pl.when(i >= 2)
                def _():
                    wait_write(b)

                transpose(ins[b], outs[b])
                start_write(i, b)

                @pl.when(i + 2 < STEPS_A)
                def _():
                    start_read(i + 2, b)

            return carry

        lax.fori_loop(0, STEPS_A // 2, body, 0)

        i = STEPS_A - 1
        b = i % 2
        wait_read(b)
        wait_write(b)
        transpose(ins[b], outs[b])
        start_write(i, b)
        wait_write(1 - b)
        wait_write(b)

        @pl.when(wid == 0)
        def _():
            tbuf = out0.at[pl.ds(0, TAIL), :]
            pltpu.async_copy(tail_hbm, tbuf, os0)
            pltpu.make_async_copy(tail_hbm, tbuf, os0).wait()
            pltpu.async_copy(
                tbuf, tpad_hbm.at[pl.ds(FULL_BLOCKS * LANES, TAIL), :], os0)
            pltpu.make_async_copy(
                tbuf, tpad_hbm.at[pl.ds(0, TAIL), :], os0).wait()

    return k(table_t, tail_p)


def _gather(tpad, idx_t):
    @functools.partial(
        pl.kernel,
        mesh=_mesh(),
        out_type=jax.ShapeDtypeStruct((SEQ, EMBED, BATCH), jnp.float32),
        scratch_types=[
            pltpu.VMEM((SEQ, BBLK), jnp.int32),
            *[pltpu.VMEM((BBLK, LANES), jnp.float32) for _ in range(3)],
            *[pltpu.VMEM((EMBED, BBLK), jnp.float32) for _ in range(3)],
            pltpu.SemaphoreType.DMA,
            *[pltpu.SemaphoreType.DMA for _ in range(3)],
            *[pltpu.SemaphoreType.DMA for _ in range(3)],
        ],
        compiler_params=pltpu.CompilerParams(needs_layout_passes=False),
    )
    def k(tpad_hbm, idx_hbm, out_hbm, idx_v, g0, g1, g2, o0, o1, o2,
          xsem, gs0, gs1, gs2, os0, os1, os2):
        gbufs, gsems = (g0, g1, g2), (gs0, gs1, gs2)
        obufs, osems = (o0, o1, o2), (os0, os1, os2)
        wid = lax.axis_index("s") * NC + lax.axis_index("c")
        b0 = wid * BBLK

        pltpu.async_copy(idx_hbm.at[:, pl.ds(b0, BBLK)], idx_v, xsem)
        pltpu.make_async_copy(
            idx_hbm.at[:, pl.ds(0, BBLK)], idx_v, xsem).wait()

        def start_gather(s, g):
            pltpu.async_copy(tpad_hbm.at[idx_v.at[s]], gbufs[g], gsems[g])

        def wait_gather(s, g):
            pltpu.make_async_copy(
                tpad_hbm.at[idx_v.at[s]], gbufs[g], gsems[g]).wait()

        rows_b = [_iota16() + 16 * j for j in range(BBLK // 16)]
        i16 = _iota16()

        def transpose(src, dst):
            for jr in range(BBLK // 16):
                rowv = rows_b[jr]

                @plsc.parallel_loop(0, EMBED, step=16, unroll=2)
                def _(e0):
                    for g in range(16):
                        colv = jnp.where(i16 < 16 - g, i16 + g,
                                         i16 + (g - 16)) + e0
                        val = plsc.load_gather(src, [rowv, colv])
                        plsc.store_scatter(dst, [colv, rowv], val)

        def start_out(s, o):
            pltpu.async_copy(obufs[o], out_hbm.at[s, :, pl.ds(b0, BBLK)],
                             osems[o])

        def wait_out(o):
            pltpu.make_async_copy(
                obufs[o], out_hbm.at[0, :, pl.ds(b0, BBLK)], osems[o]).wait()

        for s in range(3):
            start_gather(s, s)

        def body(g3, carry):
            for k3 in range(3):
                s = 3 * g3 + k3
                wait_gather(s, k3)

                @pl.when(s >= 3)
                def _():
                    wait_out(k3)

                transpose(gbufs[k3], obufs[k3])
                start_out(s, k3)

                @pl.when(s + 3 < SEQ)
                def _():
                    start_gather(s + 3, k3)

            return carry

        lax.fori_loop(0, SEQ // 3, body, 0)

        for s in (SEQ - 2, SEQ - 1):
            k3 = s % 3
            wait_gather(s, k3)
            wait_out(k3)
            transpose(gbufs[k3], obufs[k3])
            start_out(s, k3)
        for k3 in range(3):
            wait_out(k3)

    return k(tpad, idx_t)


def kernel(input, table):
    tail_p = jnp.pad(table[FULL_BLOCKS * LANES:], ((0, 0), (0, LANES - EMBED)))
    tpad = _transpose_table(table.T, tail_p)
    out_t = _gather(tpad, input.T)
    return jnp.transpose(out_t, (2, 0, 1))
